# Initial kernel scaffold; baseline (speedup 1.0000x reference)
import jax
import jax.numpy as jnp
from jax.experimental import pallas as pl


def kernel(feats, edge_index, W1, b1, Wr1, br1, g1, be1, W2, b2, Wr2, br2, g2, be2):
    raise NotImplementedError("write your pallas kernel here")


# speedup vs baseline: 4.7868x; 4.7868x over previous
import functools
import math

import jax
import jax.numpy as jnp
from jax import lax
from jax.experimental import pallas as pl
from jax.experimental.pallas import tpu as pltpu
from jax.experimental.pallas import tpu_sc as plsc

_N = 10000
_E = 320000
_D = 128
_H = 64

_NC = 2
_NS = 16
_EPT = _E // (_NC * _NS)
_C = 80
_NCHUNK = _EPT // _C
_NPAD = 10240
_RPT = _NPAD // _NS
_RZ = 32

_INV = 1.0 / math.sqrt(1.0 + 1e-5)

_ROW_BLK = 1000


def _seg_sum_body(h_hbm, src_hbm, dst_hbm, out_hbm,
                  acc, src_v, dst_v, rows_v, zbuf, sem):
    c = lax.axis_index("c")
    s = lax.axis_index("s")

    zv = jnp.zeros((16,), jnp.float32)

    def zrow(i, carry):
        for k in range(_H // 16):
            zbuf[i, pl.ds(16 * k, 16)] = zv
        return carry

    lax.fori_loop(0, _RZ, zrow, 0)

    def zslice(j, carry):
        pltpu.sync_copy(zbuf, acc.at[pl.ds(s * _RPT + j * _RZ, _RZ)])
        return carry

    lax.fori_loop(0, _RPT // _RZ, zslice, 0)
    plsc.subcore_barrier()

    base = (c * _NS + s) * _EPT

    def chunk(g, carry):
        off = base + g * _C
        pltpu.sync_copy(src_hbm.at[pl.ds(off, _C)], src_v)
        pltpu.sync_copy(dst_hbm.at[pl.ds(off, _C)], dst_v)
        pltpu.async_copy(h_hbm.at[src_v], rows_v, sem).wait()
        pltpu.sync_copy(rows_v, acc.at[dst_v], add=True)
        return carry

    lax.fori_loop(0, _NCHUNK, chunk, 0)
    plsc.subcore_barrier()

    pltpu.sync_copy(acc.at[pl.ds(s * _RPT, _RPT)],
                    out_hbm.at[c, pl.ds(s * _RPT, _RPT)])


def _seg_sum(h, src, dst):
    mesh = plsc.VectorSubcoreMesh(core_axis_name="c", subcore_axis_name="s")
    f = functools.partial(
        pl.kernel,
        mesh=mesh,
        compiler_params=pltpu.CompilerParams(use_tc_tiling_on_sc=False),
        out_type=jax.ShapeDtypeStruct((_NC, _NPAD, _H), jnp.float32),
        scratch_types=[
            pltpu.VMEM_SHARED((_NPAD, _H), jnp.float32),
            pltpu.VMEM((_C,), jnp.int32),
            pltpu.VMEM((_C,), jnp.int32),
            pltpu.VMEM((_C, _H), jnp.float32),
            pltpu.VMEM((_RZ, _H), jnp.float32),
            pltpu.SemaphoreType.DMA,
        ],
    )(_seg_sum_body)
    return f(h, src, dst)


def _lin1_body(x_ref, w_ref, wr_ref, br_ref, h_ref, r_ref):
    x = x_ref[...]
    h_ref[...] = jnp.dot(x, w_ref[...], preferred_element_type=jnp.float32)
    r_ref[...] = jnp.maximum(
        jnp.dot(x, wr_ref[...], preferred_element_type=jnp.float32)
        + br_ref[...], 0.0)


def _lin1(x, w, wr, br):
    grid = _N // _ROW_BLK
    d_in = x.shape[1]
    return pl.pallas_call(
        _lin1_body,
        grid=(grid,),
        in_specs=[
            pl.BlockSpec((_ROW_BLK, d_in), lambda i: (i, 0)),
            pl.BlockSpec((d_in, _H), lambda i: (0, 0)),
            pl.BlockSpec((d_in, _H), lambda i: (0, 0)),
            pl.BlockSpec((1, _H), lambda i: (0, 0)),
        ],
        out_specs=[
            pl.BlockSpec((_ROW_BLK, _H), lambda i: (i, 0)),
            pl.BlockSpec((_ROW_BLK, _H), lambda i: (i, 0)),
        ],
        out_shape=[
            jax.ShapeDtypeStruct((_N, _H), jnp.float32),
            jax.ShapeDtypeStruct((_N, _H), jnp.float32),
        ],
    )(x, w, wr, br.reshape(1, _H))


def _mid_body(a0_ref, a1_ref, r_ref, b_ref, g_ref, be_ref,
              w_ref, wr_ref, br_ref, h_ref, r2_ref):
    agg = a0_ref[...] + a1_ref[...]
    x = jnp.maximum(agg + b_ref[...], 0.0) + r_ref[...]
    x = g_ref[...] * (x * _INV) + be_ref[...]
    h_ref[...] = jnp.dot(x, w_ref[...], preferred_element_type=jnp.float32)
    r2_ref[...] = jnp.maximum(
        jnp.dot(x, wr_ref[...], preferred_element_type=jnp.float32)
        + br_ref[...], 0.0)


def _mid(a0, a1, r, b, g, be, w, wr, br):
    grid = _N // _ROW_BLK
    row = pl.BlockSpec((_ROW_BLK, _H), lambda i: (i, 0))
    vec = pl.BlockSpec((1, _H), lambda i: (0, 0))
    mat = pl.BlockSpec((_H, _H), lambda i: (0, 0))
    return pl.pallas_call(
        _mid_body,
        grid=(grid,),
        in_specs=[row, row, row, vec, vec, vec, mat, mat, vec],
        out_specs=[row, row],
        out_shape=[
            jax.ShapeDtypeStruct((_N, _H), jnp.float32),
            jax.ShapeDtypeStruct((_N, _H), jnp.float32),
        ],
    )(a0, a1, r, b.reshape(1, _H), g.reshape(1, _H), be.reshape(1, _H),
      w, wr, br.reshape(1, _H))


def _fin_body(a0_ref, a1_ref, r_ref, b_ref, g_ref, be_ref, o_ref):
    agg = a0_ref[...] + a1_ref[...]
    x = jnp.maximum(agg + b_ref[...], 0.0) + r_ref[...]
    o_ref[...] = g_ref[...] * (x * _INV) + be_ref[...]


def _fin(a0, a1, r, b, g, be):
    grid = _N // _ROW_BLK
    row = pl.BlockSpec((_ROW_BLK, _H), lambda i: (i, 0))
    vec = pl.BlockSpec((1, _H), lambda i: (0, 0))
    return pl.pallas_call(
        _fin_body,
        grid=(grid,),
        in_specs=[row, row, row, vec, vec, vec],
        out_specs=row,
        out_shape=jax.ShapeDtypeStruct((_N, _H), jnp.float32),
    )(a0, a1, r, b.reshape(1, _H), g.reshape(1, _H), be.reshape(1, _H))


def kernel(feats, edge_index, W1, b1, Wr1, br1, g1, be1,
           W2, b2, Wr2, br2, g2, be2):
    src = edge_index[0]
    dst = edge_index[1]

    h1, r1 = _lin1(feats, W1, Wr1, br1)
    p1 = _seg_sum(h1, src, dst)
    h2, r2 = _mid(p1[0, :_N], p1[1, :_N], r1, b1, g1, be1, W2, Wr2, br2)
    p2 = _seg_sum(h2, src, dst)
    return _fin(p2[0, :_N], p2[1, :_N], r2, b2, g2, be2)


# speedup vs baseline: 12.8525x; 2.6850x over previous
import functools
import math

import jax
import jax.numpy as jnp
from jax import lax
from jax.experimental import pallas as pl
from jax.experimental.pallas import tpu as pltpu
from jax.experimental.pallas import tpu_sc as plsc

_N = 10000
_E = 320000
_D = 128
_H = 64

_NC = 2
_NS = 16
_EPT = _E // (_NC * _NS)
_C = 80
_NCHUNK = _EPT // _C
_NPAD = 10240
_RPT = _NPAD // _NS
_RZ = 32

_INV = 1.0 / math.sqrt(1.0 + 1e-5)

_ROW_BLK = 1000


_NB = 5
_NG = _NCHUNK // _NB


def _seg_sum_body(h_hbm, src_hbm, dst_hbm, out_hbm,
                  acc, srcb, dstb, rows, zbuf, gsem, ssem):
    c = lax.axis_index("c")
    s = lax.axis_index("s")
    wid = c * _NS + s

    pltpu.sync_copy(src_hbm.at[wid], srcb)
    pltpu.sync_copy(dst_hbm.at[wid], dstb)

    zv = jnp.zeros((16,), jnp.float32)

    def zrow(i, carry):
        for k in range(_H // 16):
            zbuf[i, pl.ds(16 * k, 16)] = zv
        return carry

    lax.fori_loop(0, _RZ, zrow, 0)

    def zslice(j, carry):
        pltpu.sync_copy(zbuf, acc.at[pl.ds(s * _RPT + j * _RZ, _RZ)])
        return carry

    lax.fori_loop(0, _RPT // _RZ, zslice, 0)
    plsc.subcore_barrier()

    for b in range(_NB):
        pltpu.async_copy(h_hbm.at[srcb.at[b]], rows.at[0, b], gsem)

    def grp(i, carry):
        st = lax.rem(i, 2)
        nxt = 1 - st

        @pl.when(i >= 1)
        def _():
            for b in range(_NB):
                pltpu.make_async_copy(
                    rows.at[nxt, b], acc.at[dstb.at[b]], ssem).wait()

        @pl.when(i + 1 < _NG)
        def _():
            for b in range(_NB):
                g = (i + 1) * _NB + b
                pltpu.async_copy(h_hbm.at[srcb.at[g]], rows.at[nxt, b], gsem)

        for b in range(_NB):
            pltpu.make_async_copy(
                h_hbm.at[srcb.at[b]], rows.at[st, b], gsem).wait()
        for b in range(_NB):
            g = i * _NB + b
            pltpu.async_copy(rows.at[st, b], acc.at[dstb.at[g]], ssem,
                             add=True)
        return carry

    lax.fori_loop(0, _NG, grp, 0)
    for b in range(_NB):
        pltpu.make_async_copy(
            rows.at[(_NG - 1) % 2, b], acc.at[dstb.at[b]], ssem).wait()

    plsc.subcore_barrier()

    pltpu.sync_copy(acc.at[pl.ds(s * _RPT, _RPT)],
                    out_hbm.at[c, pl.ds(s * _RPT, _RPT)])


def _seg_sum(h, src3, dst3):
    mesh = plsc.VectorSubcoreMesh(core_axis_name="c", subcore_axis_name="s")
    f = functools.partial(
        pl.kernel,
        mesh=mesh,
        compiler_params=pltpu.CompilerParams(use_tc_tiling_on_sc=False),
        out_type=jax.ShapeDtypeStruct((_NC, _NPAD, _H), jnp.float32),
        scratch_types=[
            pltpu.VMEM_SHARED((_NPAD, _H), jnp.float32),
            pltpu.VMEM((_NCHUNK, _C), jnp.int32),
            pltpu.VMEM((_NCHUNK, _C), jnp.int32),
            pltpu.VMEM((2, _NB, _C, _H), jnp.float32),
            pltpu.VMEM((_RZ, _H), jnp.float32),
            pltpu.SemaphoreType.DMA,
            pltpu.SemaphoreType.DMA,
        ],
    )(_seg_sum_body)
    return f(h, src3, dst3)


def _lin1_body(x_ref, w_ref, wr_ref, br_ref, h_ref, r_ref):
    x = x_ref[...]
    h_ref[...] = jnp.dot(x, w_ref[...], preferred_element_type=jnp.float32)
    r_ref[...] = jnp.maximum(
        jnp.dot(x, wr_ref[...], preferred_element_type=jnp.float32)
        + br_ref[...], 0.0)


def _lin1(x, w, wr, br):
    grid = _N // _ROW_BLK
    d_in = x.shape[1]
    return pl.pallas_call(
        _lin1_body,
        grid=(grid,),
        in_specs=[
            pl.BlockSpec((_ROW_BLK, d_in), lambda i: (i, 0)),
            pl.BlockSpec((d_in, _H), lambda i: (0, 0)),
            pl.BlockSpec((d_in, _H), lambda i: (0, 0)),
            pl.BlockSpec((1, _H), lambda i: (0, 0)),
        ],
        out_specs=[
            pl.BlockSpec((_ROW_BLK, _H), lambda i: (i, 0)),
            pl.BlockSpec((_ROW_BLK, _H), lambda i: (i, 0)),
        ],
        out_shape=[
            jax.ShapeDtypeStruct((_N, _H), jnp.float32),
            jax.ShapeDtypeStruct((_N, _H), jnp.float32),
        ],
    )(x, w, wr, br.reshape(1, _H))


def _mid_body(a0_ref, a1_ref, r_ref, b_ref, g_ref, be_ref,
              w_ref, wr_ref, br_ref, h_ref, r2_ref):
    agg = a0_ref[...] + a1_ref[...]
    x = jnp.maximum(agg + b_ref[...], 0.0) + r_ref[...]
    x = g_ref[...] * (x * _INV) + be_ref[...]
    h_ref[...] = jnp.dot(x, w_ref[...], preferred_element_type=jnp.float32)
    r2_ref[...] = jnp.maximum(
        jnp.dot(x, wr_ref[...], preferred_element_type=jnp.float32)
        + br_ref[...], 0.0)


def _mid(a0, a1, r, b, g, be, w, wr, br):
    grid = _N // _ROW_BLK
    row = pl.BlockSpec((_ROW_BLK, _H), lambda i: (i, 0))
    vec = pl.BlockSpec((1, _H), lambda i: (0, 0))
    mat = pl.BlockSpec((_H, _H), lambda i: (0, 0))
    return pl.pallas_call(
        _mid_body,
        grid=(grid,),
        in_specs=[row, row, row, vec, vec, vec, mat, mat, vec],
        out_specs=[row, row],
        out_shape=[
            jax.ShapeDtypeStruct((_N, _H), jnp.float32),
            jax.ShapeDtypeStruct((_N, _H), jnp.float32),
        ],
    )(a0, a1, r, b.reshape(1, _H), g.reshape(1, _H), be.reshape(1, _H),
      w, wr, br.reshape(1, _H))


def _fin_body(a0_ref, a1_ref, r_ref, b_ref, g_ref, be_ref, o_ref):
    agg = a0_ref[...] + a1_ref[...]
    x = jnp.maximum(agg + b_ref[...], 0.0) + r_ref[...]
    o_ref[...] = g_ref[...] * (x * _INV) + be_ref[...]


def _fin(a0, a1, r, b, g, be):
    grid = _N // _ROW_BLK
    row = pl.BlockSpec((_ROW_BLK, _H), lambda i: (i, 0))
    vec = pl.BlockSpec((1, _H), lambda i: (0, 0))
    return pl.pallas_call(
        _fin_body,
        grid=(grid,),
        in_specs=[row, row, row, vec, vec, vec],
        out_specs=row,
        out_shape=jax.ShapeDtypeStruct((_N, _H), jnp.float32),
    )(a0, a1, r, b.reshape(1, _H), g.reshape(1, _H), be.reshape(1, _H))


def kernel(feats, edge_index, W1, b1, Wr1, br1, g1, be1,
           W2, b2, Wr2, br2, g2, be2):
    src3 = edge_index[0].reshape(_NC * _NS, _NCHUNK, _C)
    dst3 = edge_index[1].reshape(_NC * _NS, _NCHUNK, _C)

    h1, r1 = _lin1(feats, W1, Wr1, br1)
    p1 = _seg_sum(h1, src3, dst3)
    h2, r2 = _mid(p1[0, :_N], p1[1, :_N], r1, b1, g1, be1, W2, Wr2, br2)
    p2 = _seg_sum(h2, src3, dst3)
    return _fin(p2[0, :_N], p2[1, :_N], r2, b2, g2, be2)


# speedup vs baseline: 14.0567x; 1.0937x over previous
import functools
import math

import jax
import jax.numpy as jnp
from jax import lax
from jax.experimental import pallas as pl
from jax.experimental.pallas import tpu as pltpu
from jax.experimental.pallas import tpu_sc as plsc

_N = 10000
_E = 320000
_D = 128
_H = 64

_NC = 2
_NS = 16
_EPT = _E // (_NC * _NS)
_C = 80
_NCHUNK = _EPT // _C
_NPAD = 10240
_RPT = _NPAD // _NS
_RZ = 32

_INV = 1.0 / math.sqrt(1.0 + 1e-5)

_ROW_BLK = 1000


_NB = 5
_NG = _NCHUNK // _NB


def _seg_sum_body(h_hbm, src_hbm, dst_hbm, out0_hbm, out1_hbm,
                  acc, srcb, dstb, rows, zbuf, gsem, ssem):
    c = lax.axis_index("c")
    s = lax.axis_index("s")
    wid = c * _NS + s

    pltpu.sync_copy(src_hbm.at[wid], srcb)
    pltpu.sync_copy(dst_hbm.at[wid], dstb)

    zv = jnp.zeros((16,), jnp.float32)

    def zrow(i, carry):
        for k in range(_H // 16):
            zbuf[i, pl.ds(16 * k, 16)] = zv
        return carry

    lax.fori_loop(0, _RZ, zrow, 0)

    def zslice(j, carry):
        pltpu.sync_copy(zbuf, acc.at[pl.ds(s * _RPT + j * _RZ, _RZ)])
        return carry

    lax.fori_loop(0, _RPT // _RZ, zslice, 0)
    plsc.subcore_barrier()

    for b in range(_NB):
        pltpu.async_copy(h_hbm.at[srcb.at[b]], rows.at[0, b], gsem)

    def grp(i, carry):
        st = lax.rem(i, 2)
        nxt = 1 - st

        @pl.when(i >= 1)
        def _():
            for b in range(_NB):
                pltpu.make_async_copy(
                    rows.at[nxt, b], acc.at[dstb.at[b]], ssem).wait()

        @pl.when(i + 1 < _NG)
        def _():
            for b in range(_NB):
                g = (i + 1) * _NB + b
                pltpu.async_copy(h_hbm.at[srcb.at[g]], rows.at[nxt, b], gsem)

        for b in range(_NB):
            pltpu.make_async_copy(
                h_hbm.at[srcb.at[b]], rows.at[st, b], gsem).wait()
        for b in range(_NB):
            g = i * _NB + b
            pltpu.async_copy(rows.at[st, b], acc.at[dstb.at[g]], ssem,
                             add=True)
        return carry

    lax.fori_loop(0, _NG, grp, 0)
    for b in range(_NB):
        pltpu.make_async_copy(
            rows.at[(_NG - 1) % 2, b], acc.at[dstb.at[b]], ssem).wait()

    plsc.subcore_barrier()

    @pl.when(c == 0)
    def _():
        pltpu.sync_copy(acc.at[pl.ds(s * _RPT, _RPT)],
                        out0_hbm.at[pl.ds(s * _RPT, _RPT)])

    @pl.when(c == 1)
    def _():
        pltpu.sync_copy(acc.at[pl.ds(s * _RPT, _RPT)],
                        out1_hbm.at[pl.ds(s * _RPT, _RPT)])


def _seg_sum(h, src3, dst3):
    mesh = plsc.VectorSubcoreMesh(core_axis_name="c", subcore_axis_name="s")
    f = functools.partial(
        pl.kernel,
        mesh=mesh,
        compiler_params=pltpu.CompilerParams(use_tc_tiling_on_sc=False),
        out_type=[jax.ShapeDtypeStruct((_NPAD, _H), jnp.float32),
                  jax.ShapeDtypeStruct((_NPAD, _H), jnp.float32)],
        scratch_types=[
            pltpu.VMEM_SHARED((_NPAD, _H), jnp.float32),
            pltpu.VMEM((_NCHUNK, _C), jnp.int32),
            pltpu.VMEM((_NCHUNK, _C), jnp.int32),
            pltpu.VMEM((2, _NB, _C, _H), jnp.float32),
            pltpu.VMEM((_RZ, _H), jnp.float32),
            pltpu.SemaphoreType.DMA,
            pltpu.SemaphoreType.DMA,
        ],
    )(_seg_sum_body)
    return f(h, src3, dst3)


def _matmul_body(x_ref, w_ref, h_ref):
    h_ref[...] = jnp.dot(x_ref[...], w_ref[...],
                         preferred_element_type=jnp.float32)


def _matmul(x, w):
    grid = _N // _ROW_BLK
    d_in = x.shape[1]
    return pl.pallas_call(
        _matmul_body,
        grid=(grid,),
        in_specs=[
            pl.BlockSpec((_ROW_BLK, d_in), lambda i: (i, 0)),
            pl.BlockSpec((d_in, _H), lambda i: (0, 0)),
        ],
        out_specs=pl.BlockSpec((_ROW_BLK, _H), lambda i: (i, 0)),
        out_shape=jax.ShapeDtypeStruct((_N, _H), jnp.float32),
    )(x, w)


def _res_body(x_ref, wr_ref, br_ref, r_ref):
    r_ref[...] = jnp.maximum(
        jnp.dot(x_ref[...], wr_ref[...], preferred_element_type=jnp.float32)
        + br_ref[...], 0.0)


def _res(x, wr, br):
    grid = _N // _ROW_BLK
    d_in = x.shape[1]
    return pl.pallas_call(
        _res_body,
        grid=(grid,),
        in_specs=[
            pl.BlockSpec((_ROW_BLK, d_in), lambda i: (i, 0)),
            pl.BlockSpec((d_in, _H), lambda i: (0, 0)),
            pl.BlockSpec((1, _H), lambda i: (0, 0)),
        ],
        out_specs=pl.BlockSpec((_ROW_BLK, _H), lambda i: (i, 0)),
        out_shape=jax.ShapeDtypeStruct((_N, _H), jnp.float32),
    )(x, wr, br.reshape(1, _H))


def _mid_body(a0_ref, a1_ref, r_ref, b_ref, g_ref, be_ref,
              w_ref, h_ref, x_ref):
    agg = a0_ref[...] + a1_ref[...]
    x = jnp.maximum(agg + b_ref[...], 0.0) + r_ref[...]
    x = g_ref[...] * (x * _INV) + be_ref[...]
    h_ref[...] = jnp.dot(x, w_ref[...], preferred_element_type=jnp.float32)
    x_ref[...] = x


def _mid(a0, a1, r, b, g, be, w):
    grid = _N // _ROW_BLK
    row = pl.BlockSpec((_ROW_BLK, _H), lambda i: (i, 0))
    vec = pl.BlockSpec((1, _H), lambda i: (0, 0))
    mat = pl.BlockSpec((_H, _H), lambda i: (0, 0))
    return pl.pallas_call(
        _mid_body,
        grid=(grid,),
        in_specs=[row, row, row, vec, vec, vec, mat],
        out_specs=[row, row],
        out_shape=[
            jax.ShapeDtypeStruct((_N, _H), jnp.float32),
            jax.ShapeDtypeStruct((_N, _H), jnp.float32),
        ],
    )(a0, a1, r, b.reshape(1, _H), g.reshape(1, _H), be.reshape(1, _H), w)


def _fin_body(a0_ref, a1_ref, r_ref, b_ref, g_ref, be_ref, o_ref):
    agg = a0_ref[...] + a1_ref[...]
    x = jnp.maximum(agg + b_ref[...], 0.0) + r_ref[...]
    o_ref[...] = g_ref[...] * (x * _INV) + be_ref[...]


def _fin(a0, a1, r, b, g, be):
    grid = _N // _ROW_BLK
    row = pl.BlockSpec((_ROW_BLK, _H), lambda i: (i, 0))
    vec = pl.BlockSpec((1, _H), lambda i: (0, 0))
    return pl.pallas_call(
        _fin_body,
        grid=(grid,),
        in_specs=[row, row, row, vec, vec, vec],
        out_specs=row,
        out_shape=jax.ShapeDtypeStruct((_N, _H), jnp.float32),
    )(a0, a1, r, b.reshape(1, _H), g.reshape(1, _H), be.reshape(1, _H))


def kernel(feats, edge_index, W1, b1, Wr1, br1, g1, be1,
           W2, b2, Wr2, br2, g2, be2):
    src3 = edge_index[0].reshape(_NC * _NS, _NCHUNK, _C)
    dst3 = edge_index[1].reshape(_NC * _NS, _NCHUNK, _C)

    h1 = _matmul(feats, W1)
    a10, a11 = _seg_sum(h1, src3, dst3)
    r1 = _res(feats, Wr1, br1)
    h2, x2 = _mid(a10, a11, r1, b1, g1, be1, W2)
    a20, a21 = _seg_sum(h2, src3, dst3)
    r2 = _res(x2, Wr2, br2)
    return _fin(a20, a21, r2, b2, g2, be2)
